# Initial kernel scaffold; baseline (speedup 1.0000x reference)
import jax
import jax.numpy as jnp
from jax.experimental import pallas as pl
from jax.experimental.pallas import tpu as pltpu

_N = 4096
_D = 64
_NP = 4
_NA = 32
_H = 64
_TOPK = 32
_ZD = _NP * _D + _H
_BLOCK = 256


def _z_kernel(ctx_ref, pe_ref, w_ref, wpe_ref, sa_ref, sb_ref, z_ref):
    ctx = ctx_ref[...]
    w = w_ref[...]
    sa = sa_ref[0, 0]
    sb = sb_ref[0, 0]
    for p in range(_NP):
        x = ctx * w[p, :][None, :]
        nrm = jnp.sqrt(jnp.sum(x * x, axis=1, keepdims=True))
        x = x / jnp.maximum(nrm, 1e-12)
        z_ref[:, p * _D:(p + 1) * _D] = x * sa
    pe = jax.lax.dot_general(
        pe_ref[...], wpe_ref[...], (((1,), (0,)), ((), ())),
        preferred_element_type=jnp.float32)
    z_ref[:, _NP * _D:] = pe * sb


def _topk_kernel(zrow_ref, zall_ref, gpr_ref, out_ref, v_ref):
    s = jax.lax.dot_general(
        zrow_ref[...], zall_ref[...], (((1,), (1,)), ((), ())),
        preferred_element_type=jnp.float32)
    s = s * gpr_ref[...]
    out_ref[...] = s
    v_ref[...] = s

    def body(_, t):
        v = v_ref[...]
        m = jnp.max(v, axis=1, keepdims=True)
        v_ref[...] = jnp.where(v == m, -jnp.inf, v)
        return m

    t = jax.lax.fori_loop(
        0, _TOPK, body, jnp.full((_BLOCK, 1), -jnp.inf, jnp.float32))
    s = out_ref[...]
    out_ref[...] = jnp.where(s >= t, s, 0.0)


def kernel(context, position_encoding, gpr_rank, position_flag, W, Wpe):
    flag = jnp.asarray(position_flag)
    ca = jnp.where(flag == 1, 0.5, 1.0).astype(jnp.float32)
    cb = jnp.where(flag == 1, 0.5, 0.0).astype(jnp.float32)
    sa = jnp.sqrt(ca / _NP).reshape(1, 1)
    sb = jnp.sqrt(cb).reshape(1, 1)

    z = pl.pallas_call(
        _z_kernel,
        out_shape=jax.ShapeDtypeStruct((_N, _ZD), jnp.float32),
    )(context, position_encoding, W, Wpe, sa, sb)

    out = pl.pallas_call(
        _topk_kernel,
        grid=(_N // _BLOCK,),
        in_specs=[
            pl.BlockSpec((_BLOCK, _ZD), lambda i: (i, 0)),
            pl.BlockSpec((_N, _ZD), lambda i: (0, 0)),
            pl.BlockSpec((_BLOCK, 1), lambda i: (i, 0)),
        ],
        out_specs=pl.BlockSpec((_BLOCK, _N), lambda i: (i, 0)),
        out_shape=jax.ShapeDtypeStruct((_N, _N), jnp.float32),
        scratch_shapes=[pltpu.VMEM((_BLOCK, _N), jnp.float32)],
        compiler_params=pltpu.CompilerParams(
            dimension_semantics=("arbitrary",)),
    )(z, z, gpr_rank)
    return out


# speedup vs baseline: 20.8581x; 20.8581x over previous
import jax
import jax.numpy as jnp
from jax.experimental import pallas as pl
from jax.experimental.pallas import tpu as pltpu

_N = 4096
_D = 64
_NP = 4
_NA = 32
_H = 64
_TOPK = 32
_ZD = _NP * _D + _H
_BLOCK = 256


def _z_kernel(ctx_ref, pe_ref, w_ref, wpe_ref, z_ref):
    ctx = ctx_ref[...]
    w = w_ref[...]
    for p in range(_NP):
        x = ctx * w[p, :][None, :]
        nrm = jnp.sqrt(jnp.sum(x * x, axis=1, keepdims=True))
        x = x / jnp.maximum(nrm, 1e-12)
        z_ref[:, p * _D:(p + 1) * _D] = x
    pe = jax.lax.dot_general(
        pe_ref[...], wpe_ref[...], (((1,), (0,)), ((), ())),
        preferred_element_type=jnp.float32)
    z_ref[:, _NP * _D:] = pe


def _topk_kernel(zrow_ref, zall_ref, gpr_ref, wa_ref, wb_ref,
                 out_ref, v_ref):
    zr = zrow_ref[...]
    za = zall_ref[...]
    dn = (((1,), (1,)), ((), ()))
    acc = jax.lax.dot_general(
        zr[:, 0:_D], za[:, 0:_D], dn, preferred_element_type=jnp.float32)
    for p in range(1, _NP):
        acc = acc + jax.lax.dot_general(
            zr[:, p * _D:(p + 1) * _D], za[:, p * _D:(p + 1) * _D], dn,
            preferred_element_type=jnp.float32)
    mean_att = acc * (1.0 / _NP)
    pe_att = jax.lax.dot_general(
        zr[:, _NP * _D:], za[:, _NP * _D:], dn,
        preferred_element_type=jnp.float32)
    s = (wa_ref[0, 0] * mean_att + wb_ref[0, 0] * pe_att)
    s = s * gpr_ref[...]
    out_ref[...] = s
    v_ref[...] = s

    def body(_, t):
        v = v_ref[...]
        m = jnp.max(v, axis=1, keepdims=True)
        v_ref[...] = jnp.where(v == m, -jnp.inf, v)
        return m

    t = jax.lax.fori_loop(
        0, _TOPK, body, jnp.full((_BLOCK, 1), -jnp.inf, jnp.float32))
    s = out_ref[...]
    out_ref[...] = jnp.where(s >= t, s, 0.0)


def kernel(context, position_encoding, gpr_rank, position_flag, W, Wpe):
    flag = jnp.asarray(position_flag)
    wa = jnp.where(flag == 1, 0.5, 1.0).astype(jnp.float32).reshape(1, 1)
    wb = jnp.where(flag == 1, 0.5, 0.0).astype(jnp.float32).reshape(1, 1)

    z = pl.pallas_call(
        _z_kernel,
        out_shape=jax.ShapeDtypeStruct((_N, _ZD), jnp.float32),
    )(context, position_encoding, W, Wpe)

    out = pl.pallas_call(
        _topk_kernel,
        grid=(_N // _BLOCK,),
        in_specs=[
            pl.BlockSpec((_BLOCK, _ZD), lambda i: (i, 0)),
            pl.BlockSpec((_N, _ZD), lambda i: (0, 0)),
            pl.BlockSpec((_BLOCK, 1), lambda i: (i, 0)),
            pl.BlockSpec((1, 1), lambda i: (0, 0)),
            pl.BlockSpec((1, 1), lambda i: (0, 0)),
        ],
        out_specs=pl.BlockSpec((_BLOCK, _N), lambda i: (i, 0)),
        out_shape=jax.ShapeDtypeStruct((_N, _N), jnp.float32),
        scratch_shapes=[pltpu.VMEM((_BLOCK, _N), jnp.float32)],
        compiler_params=pltpu.CompilerParams(
            dimension_semantics=("arbitrary",)),
    )(z, z, gpr_rank, wa, wb)
    return out
